# baseline (speedup 1.0000x reference)
import functools
import math

import jax
import jax.numpy as jnp
from jax import lax
from jax.experimental import pallas as pl
from jax.experimental.pallas import tpu as pltpu
from jax.experimental.pallas import tpu_sc as plsc

_EPS = 1e-08
_LOG_LO = math.log(1e-10)
_LOG_HI = math.log(1e4)
_DIM = 64
_NB = 1024
_CHUNK = 128
_HALF = 500224
_INTERPRET = False


def _lv(w):
    sp = jnp.maximum(w, 0.0) + jnp.log1p(jnp.exp(-jnp.abs(w)))
    s = jnp.sum(jnp.log(sp), axis=-1)
    return jnp.clip(s, _LOG_LO, _LOG_HI)


def _reg_sums(mn, mx):
    s1 = jnp.sum(jnp.maximum(mx - 1.0 + _EPS, 0.0))
    s2 = jnp.sum(mn * mn)
    return s1, s2


def _unpack(w):
    hi = lax.bitcast_convert_type(w & jnp.int32(-65536), jnp.float32)
    lo = lax.bitcast_convert_type(w << 16, jnp.float32)
    return hi, lo


def _compute_body(gcomb_ref, grel_ref, mask_ref, out_ref):
    step = pl.program_id(0)
    blk = gcomb_ref[...]
    m = mask_ref[...]
    w = jnp.where(m != 0, blk[:, :, _DIM:], blk[:, :, :_DIM])
    mn, dl = _unpack(w)
    mx = mn + jnp.exp(dl)
    rblk = grel_ref[...]
    rel = lax.bitcast_convert_type(rblk[:, :, :_DIM], jnp.float32)
    sc = lax.bitcast_convert_type(rblk[:, :, _DIM:], jnp.float32) + _EPS

    mn1, mx1, mn2, mx2 = mn[0], mx[0], mn[1], mx[1]
    w_i = jnp.minimum(mx1, mx2) - jnp.maximum(mn1, mn2)
    nf1 = jnp.sum(1.0 - jnp.exp(_lv(w_i) - _lv(mx1 - mn1)))
    r_nf1 = [_reg_sums(mn1, mx1), _reg_sums(mn2, mx2)]

    mn1, mx1, mn2, mx2 = mn[2], mx[2], mn[3], mx[3]
    mn3, mx3 = mn[4], mx[4]
    mni = jnp.maximum(mn1, mn2)
    mxi = jnp.minimum(mx1, mx2)
    w_i123 = jnp.minimum(mxi, mx3) - jnp.maximum(mni, mn3)
    nf2 = jnp.sum(1.0 - jnp.exp(_lv(w_i123) - _lv(mxi - mni)))
    r_nf2 = [_reg_sums(mni, mxi), _reg_sums(mn1, mx1),
             _reg_sums(mn2, mx2), _reg_sums(mn3, mx3)]

    mn1, mx1, mn2, mx2 = mn[5], mx[5], mn[6], mx[6]
    tmn = mn1 * sc[0] + rel[0]
    tmx = mx1 * sc[0] + rel[0]
    w_i = jnp.minimum(tmx, mx2) - jnp.maximum(tmn, mn2)
    nf3 = jnp.sum(1.0 - jnp.exp(_lv(w_i) - _lv(tmx - tmn)))
    r_nf3 = [_reg_sums(tmn, tmx), _reg_sums(mn1, mx1), _reg_sums(mn2, mx2)]

    mn1, mx1, mn2, mx2 = mn[7], mx[7], mn[8], mx[8]
    tmn = (mn1 - rel[1]) / sc[1]
    tmx = (mx1 - rel[1]) / sc[1]
    w_i = jnp.minimum(tmx, mx2) - jnp.maximum(tmn, mn2)
    nf4 = jnp.sum(1.0 - jnp.exp(_lv(w_i) - _lv(tmx - tmn)))
    r_nf4 = [_reg_sums(tmn, tmx), _reg_sums(mn1, mx1), _reg_sums(mn2, mx2)]

    mn1, mx1, mn2, mx2 = mn[9], mx[9], mn[10], mx[10]
    w_i = jnp.minimum(mx1, mx2) - jnp.maximum(mn1, mn2)
    dj = jnp.sum(jnp.exp(_lv(w_i) - (_lv(mx1 - mn1) + _lv(mx2 - mn2))))
    r_dj = [_reg_sums(mn1, mx1), _reg_sums(mn2, mx2)]

    mn1, mx1, mn2, mx2 = mn[11], mx[11], mn[12], mx[12]
    tmn = mn1 * sc[2] + rel[2]
    tmx = mx1 * sc[2] + rel[2]
    w_i = jnp.minimum(tmx, mx2) - jnp.maximum(tmn, mn2)
    nf3n = jnp.sum(jnp.exp(_lv(w_i) - _lv(tmx - tmn)))
    r_nf3n = [_reg_sums(tmn, tmx), _reg_sums(mn1, mx1), _reg_sums(mn2, mx2)]

    parts = [nf1, nf2, nf3, nf4, dj, nf3n]
    for s1, s2 in (r_nf1 + r_nf2 + r_nf3 + r_nf4 + r_dj + r_nf3n):
        parts.append(s1)
        parts.append(s2)
    vals = jnp.concatenate([jnp.stack(parts),
                            jnp.zeros((128 - len(parts),), jnp.float32)])

    @pl.when(step == 0)
    def _():
        out_ref[...] = vals

    @pl.when(step != 0)
    def _():
        out_ref[...] += vals


def _compute_partials(gcomb, grel, mask):
    b = gcomb.shape[1]
    grid = (b // _NB,)
    return pl.pallas_call(
        _compute_body,
        grid=grid,
        in_specs=[
            pl.BlockSpec((13, _NB, 2 * _DIM), lambda i: (0, i, 0)),
            pl.BlockSpec((3, _NB, 2 * _DIM), lambda i: (0, i, 0)),
            pl.BlockSpec((13, _NB, _DIM), lambda i: (0, i, 0)),
        ],
        out_specs=pl.BlockSpec((128,), lambda i: (0,)),
        out_shape=jax.ShapeDtypeStruct((128,), jnp.float32),
        interpret=_INTERPRET,
    )(gcomb, grel, mask)


_VB = 512


def _pack(a, b):
    ai = lax.bitcast_convert_type(a, jnp.int32) + 0x8000
    bi = lax.bitcast_convert_type(b, jnp.int32) + 0x8000
    return (ai & jnp.int32(-65536)) | lax.shift_right_logical(bi, 16)


def _reformat_body(mlo_ref, dlo_ref, mhi_ref, dhi_ref, out_ref):
    eye = jnp.eye(_DIM, dtype=jnp.float32)
    dn = (((0,), (0,)), ((), ()))

    def t(ref):
        return lax.dot_general(ref[...], eye, dn,
                               preferred_element_type=jnp.float32)

    lo = _pack(t(mlo_ref), t(dlo_ref))
    hi = _pack(t(mhi_ref), t(dhi_ref))
    out_ref[...] = jnp.concatenate([lo, hi], axis=1)


def _build_combined(min_t, delta_t):
    n_hi = _HALF // _VB

    def lo_map(i):
        return (0, i)

    def hi_map(i):
        return (0, i + n_hi)

    return pl.pallas_call(
        _reformat_body,
        grid=(n_hi,),
        in_specs=[
            pl.BlockSpec((_DIM, _VB), lo_map),
            pl.BlockSpec((_DIM, _VB), lo_map),
            pl.BlockSpec((_DIM, _VB), hi_map),
            pl.BlockSpec((_DIM, _VB), hi_map),
        ],
        out_specs=pl.BlockSpec((_VB, 2 * _DIM), lambda i: (i, 0)),
        out_shape=jax.ShapeDtypeStruct((_HALF, 2 * _DIM), jnp.int32),
        interpret=_INTERPRET,
    )(min_t, delta_t, min_t, delta_t)


def _sc_gather_all(tcomb, trelc, idx_all, idx_rel):
    n_big = idx_all.shape[0]
    n_rel = idx_rel.shape[0]
    w = 2 * _DIM
    info = plsc.get_sparse_core_info()
    nc, ns = info.num_cores, info.num_subcores
    nw = nc * ns
    pw_big = n_big // nw
    pw_rel = n_rel // nw
    out_t = [jax.ShapeDtypeStruct((n_big, w), jnp.int32),
             jax.ShapeDtypeStruct((n_rel, w), jnp.int32)]
    mesh = plsc.VectorSubcoreMesh(core_axis_name="c", subcore_axis_name="s")

    @functools.partial(
        pl.kernel, mesh=mesh, out_type=out_t,
        scratch_types=[
            pltpu.VMEM((pw_big,), jnp.int32),
            pltpu.VMEM((pw_rel,), jnp.int32),
            pltpu.VMEM((_CHUNK, w), jnp.int32),
            pltpu.VMEM((_CHUNK, w), jnp.int32),
            pltpu.SemaphoreType.DMA,
            pltpu.SemaphoreType.DMA,
        ],
    )
    def k(tc_h, tr_h, ia_h, ir_h, ocomb, orel,
          ia_v, ir_v, b0, b1, sem_a, sem_b):
        wid = lax.axis_index("s") * nc + lax.axis_index("c")
        pltpu.sync_copy(ia_h.at[pl.ds(wid * pw_big, pw_big)], ia_v)
        pltpu.sync_copy(ir_h.at[pl.ds(wid * pw_rel, pw_rel)], ir_v)

        def pair_loop(n_pairs, tab, idx_v, out, base):
            def body(i, carry):
                c0 = i * 2 * _CHUNK
                c1 = c0 + _CHUNK
                g0 = pltpu.async_copy(tab.at[idx_v.at[pl.ds(c0, _CHUNK)]],
                                      b0, sem_a)
                g1 = pltpu.async_copy(tab.at[idx_v.at[pl.ds(c1, _CHUNK)]],
                                      b1, sem_b)
                g0.wait()
                pltpu.sync_copy(b0, out.at[pl.ds(base + c0, _CHUNK)])
                g1.wait()
                pltpu.sync_copy(b1, out.at[pl.ds(base + c1, _CHUNK)])
                return carry

            lax.fori_loop(0, n_pairs, body, 0)

        pair_loop(pw_big // (2 * _CHUNK), tc_h, ia_v, ocomb, wid * pw_big)
        pair_loop(pw_rel // (2 * _CHUNK), tr_h, ir_v, orel, wid * pw_rel)

    return k(tcomb, trelc, idx_all, idx_rel)


def kernel(min_embedding, delta_embedding, relation_embedding, scaling_embedding,
           data0, data1, data2, data3, data4, data5, data6):
    b = data0.shape[0]
    idx_all = jnp.concatenate([
        data0[:, 0], data0[:, 2],
        data1[:, 0], data1[:, 1], data1[:, 2],
        data2[:, 0], data2[:, 2],
        data3[:, 1], data3[:, 2],
        data4[:, 0], data4[:, 1],
        data6[:, 0], data6[:, 2],
    ])
    idx_rel = jnp.concatenate([data2[:, 1], data3[:, 0], data6[:, 1]])

    tcomb = _build_combined(min_embedding.T, delta_embedding.T)
    trelc = lax.bitcast_convert_type(
        jnp.concatenate([relation_embedding, scaling_embedding], axis=1),
        jnp.int32)

    hi_side = idx_all >= _HALF
    phys = jnp.where(hi_side, idx_all - _HALF, idx_all)
    mask = jnp.broadcast_to(
        hi_side.astype(jnp.int8).reshape(13, b, 1), (13, b, _DIM))

    gcomb, grelc = _sc_gather_all(tcomb, trelc, phys, idx_rel)
    gcomb = gcomb.reshape(13, b, 2 * _DIM)
    grelc = grelc.reshape(3, b, 2 * _DIM)

    s = _compute_partials(gcomb, grelc, mask)

    denom = float(b * _DIM)

    def l2s(j):
        s1 = s[6 + 2 * j]
        s2 = s[7 + 2 * j]
        return s1 / denom + jnp.maximum(jnp.sqrt(s2) - 1.0, 0.0)

    nf1_reg = l2s(0) + l2s(1)
    nf2_reg = l2s(2) + l2s(3) + l2s(4) + l2s(5)
    nf3_reg = l2s(6) + l2s(7) + l2s(8)
    nf4_reg = l2s(9) + l2s(10) + l2s(11)
    dj_reg = l2s(12) + l2s(13)
    nf3n_reg = l2s(14) + l2s(15) + l2s(16)
    return (s[0], s[1], s[2], s[3], s[4], s[5],
            nf1_reg, nf2_reg, nf3_reg, nf4_reg, dj_reg, nf3n_reg)


# speedup vs baseline: 1.6092x; 1.6092x over previous
import functools
import math

import jax
import jax.numpy as jnp
from jax import lax
from jax.experimental import pallas as pl
from jax.experimental.pallas import tpu as pltpu
from jax.experimental.pallas import tpu_sc as plsc

_EPS = 1e-08
_LOG_LO = math.log(1e-10)
_LOG_HI = math.log(1e4)
_DIM = 64
_NB = 1024
_CHUNK = 128
_INTERPRET = False


def _lv(w):
    sp = jnp.maximum(w, 0.0) + jnp.log1p(jnp.exp(-jnp.abs(w)))
    s = jnp.sum(jnp.log(sp), axis=-1)
    return jnp.clip(s, _LOG_LO, _LOG_HI)


def _reg_sums(mn, mx):
    s1 = jnp.sum(jnp.maximum(mx - 1.0 + _EPS, 0.0))
    s2 = jnp.sum(mn * mn)
    return s1, s2


def _compute_body(gcomb_ref, grel_ref, out_ref):
    step = pl.program_id(0)
    blk = gcomb_ref[...]
    mn = blk[:, :, :_DIM]
    mx = mn + jnp.exp(blk[:, :, _DIM:])
    rblk = grel_ref[...]
    rel = rblk[:, :, :_DIM]
    sc = rblk[:, :, _DIM:] + _EPS

    mn1, mx1, mn2, mx2 = mn[0], mx[0], mn[1], mx[1]
    w_i = jnp.minimum(mx1, mx2) - jnp.maximum(mn1, mn2)
    nf1 = jnp.sum(1.0 - jnp.exp(_lv(w_i) - _lv(mx1 - mn1)))
    r_nf1 = [_reg_sums(mn1, mx1), _reg_sums(mn2, mx2)]

    mn1, mx1, mn2, mx2 = mn[2], mx[2], mn[3], mx[3]
    mn3, mx3 = mn[4], mx[4]
    mni = jnp.maximum(mn1, mn2)
    mxi = jnp.minimum(mx1, mx2)
    w_i123 = jnp.minimum(mxi, mx3) - jnp.maximum(mni, mn3)
    nf2 = jnp.sum(1.0 - jnp.exp(_lv(w_i123) - _lv(mxi - mni)))
    r_nf2 = [_reg_sums(mni, mxi), _reg_sums(mn1, mx1),
             _reg_sums(mn2, mx2), _reg_sums(mn3, mx3)]

    mn1, mx1, mn2, mx2 = mn[5], mx[5], mn[6], mx[6]
    tmn = mn1 * sc[0] + rel[0]
    tmx = mx1 * sc[0] + rel[0]
    w_i = jnp.minimum(tmx, mx2) - jnp.maximum(tmn, mn2)
    nf3 = jnp.sum(1.0 - jnp.exp(_lv(w_i) - _lv(tmx - tmn)))
    r_nf3 = [_reg_sums(tmn, tmx), _reg_sums(mn1, mx1), _reg_sums(mn2, mx2)]

    mn1, mx1, mn2, mx2 = mn[7], mx[7], mn[8], mx[8]
    tmn = (mn1 - rel[1]) / sc[1]
    tmx = (mx1 - rel[1]) / sc[1]
    w_i = jnp.minimum(tmx, mx2) - jnp.maximum(tmn, mn2)
    nf4 = jnp.sum(1.0 - jnp.exp(_lv(w_i) - _lv(tmx - tmn)))
    r_nf4 = [_reg_sums(tmn, tmx), _reg_sums(mn1, mx1), _reg_sums(mn2, mx2)]

    mn1, mx1, mn2, mx2 = mn[9], mx[9], mn[10], mx[10]
    w_i = jnp.minimum(mx1, mx2) - jnp.maximum(mn1, mn2)
    dj = jnp.sum(jnp.exp(_lv(w_i) - (_lv(mx1 - mn1) + _lv(mx2 - mn2))))
    r_dj = [_reg_sums(mn1, mx1), _reg_sums(mn2, mx2)]

    mn1, mx1, mn2, mx2 = mn[11], mx[11], mn[12], mx[12]
    tmn = mn1 * sc[2] + rel[2]
    tmx = mx1 * sc[2] + rel[2]
    w_i = jnp.minimum(tmx, mx2) - jnp.maximum(tmn, mn2)
    nf3n = jnp.sum(jnp.exp(_lv(w_i) - _lv(tmx - tmn)))
    r_nf3n = [_reg_sums(tmn, tmx), _reg_sums(mn1, mx1), _reg_sums(mn2, mx2)]

    parts = [nf1, nf2, nf3, nf4, dj, nf3n]
    for s1, s2 in (r_nf1 + r_nf2 + r_nf3 + r_nf4 + r_dj + r_nf3n):
        parts.append(s1)
        parts.append(s2)
    vals = jnp.concatenate([jnp.stack(parts),
                            jnp.zeros((128 - len(parts),), jnp.float32)])

    @pl.when(step == 0)
    def _():
        out_ref[...] = vals

    @pl.when(step != 0)
    def _():
        out_ref[...] += vals


def _compute_partials(gcomb, grel):
    b = gcomb.shape[1]
    grid = (b // _NB,)
    return pl.pallas_call(
        _compute_body,
        grid=grid,
        in_specs=[
            pl.BlockSpec((13, _NB, 2 * _DIM), lambda i: (0, i, 0)),
            pl.BlockSpec((3, _NB, 2 * _DIM), lambda i: (0, i, 0)),
        ],
        out_specs=pl.BlockSpec((128,), lambda i: (0,)),
        out_shape=jax.ShapeDtypeStruct((128,), jnp.float32),
        interpret=_INTERPRET,
    )(gcomb, grel)


_VB = 4096


def _reformat_body(mint_ref, delt_ref, out_ref):
    eye = jnp.eye(_DIM, dtype=jnp.float32)
    dn = (((0,), (0,)), ((), ()))
    mt = lax.dot_general(mint_ref[...], eye, dn,
                         preferred_element_type=jnp.float32)
    dt = lax.dot_general(delt_ref[...], eye, dn,
                         preferred_element_type=jnp.float32)
    out_ref[...] = jnp.concatenate([mt, dt], axis=1)


def _build_combined(min_t, delta_t):
    v = min_t.shape[1]
    grid = (pl.cdiv(v, _VB),)
    return pl.pallas_call(
        _reformat_body,
        grid=grid,
        in_specs=[
            pl.BlockSpec((_DIM, _VB), lambda i: (0, i)),
            pl.BlockSpec((_DIM, _VB), lambda i: (0, i)),
        ],
        out_specs=pl.BlockSpec((_VB, 2 * _DIM), lambda i: (i, 0)),
        out_shape=jax.ShapeDtypeStruct((v, 2 * _DIM), jnp.float32),
        interpret=_INTERPRET,
    )(min_t, delta_t)


def _sc_gather_all(tcomb, trelc, idx_all, idx_rel):
    n_big = idx_all.shape[0]
    n_rel = idx_rel.shape[0]
    w = 2 * _DIM
    info = plsc.get_sparse_core_info()
    nc, ns = info.num_cores, info.num_subcores
    nw = nc * ns
    pw_big = n_big // nw
    pw_rel = n_rel // nw
    out_t = [jax.ShapeDtypeStruct((n_big, w), jnp.float32),
             jax.ShapeDtypeStruct((n_rel, w), jnp.float32)]
    mesh = plsc.VectorSubcoreMesh(core_axis_name="c", subcore_axis_name="s")

    @functools.partial(
        pl.kernel, mesh=mesh, out_type=out_t,
        scratch_types=[
            pltpu.VMEM((pw_big,), jnp.int32),
            pltpu.VMEM((pw_rel,), jnp.int32),
            pltpu.VMEM((_CHUNK, w), jnp.float32),
            pltpu.VMEM((_CHUNK, w), jnp.float32),
            pltpu.SemaphoreType.DMA,
            pltpu.SemaphoreType.DMA,
        ],
    )
    def k(tc_h, tr_h, ia_h, ir_h, ocomb, orel,
          ia_v, ir_v, b0, b1, sem_a, sem_b):
        wid = lax.axis_index("s") * nc + lax.axis_index("c")
        pltpu.sync_copy(ia_h.at[pl.ds(wid * pw_big, pw_big)], ia_v)
        pltpu.sync_copy(ir_h.at[pl.ds(wid * pw_rel, pw_rel)], ir_v)

        def pair_loop(n_pairs, tab, idx_v, out, base):
            def body(i, carry):
                c0 = i * 2 * _CHUNK
                c1 = c0 + _CHUNK
                g0 = pltpu.async_copy(tab.at[idx_v.at[pl.ds(c0, _CHUNK)]],
                                      b0, sem_a)
                g1 = pltpu.async_copy(tab.at[idx_v.at[pl.ds(c1, _CHUNK)]],
                                      b1, sem_b)
                g0.wait()
                pltpu.sync_copy(b0, out.at[pl.ds(base + c0, _CHUNK)])
                g1.wait()
                pltpu.sync_copy(b1, out.at[pl.ds(base + c1, _CHUNK)])
                return carry

            lax.fori_loop(0, n_pairs, body, 0)

        pair_loop(pw_big // (2 * _CHUNK), tc_h, ia_v, ocomb, wid * pw_big)
        pair_loop(pw_rel // (2 * _CHUNK), tr_h, ir_v, orel, wid * pw_rel)

    return k(tcomb, trelc, idx_all, idx_rel)


def kernel(min_embedding, delta_embedding, relation_embedding, scaling_embedding,
           data0, data1, data2, data3, data4, data5, data6):
    b = data0.shape[0]
    idx_all = jnp.concatenate([
        data0[:, 0], data0[:, 2],
        data1[:, 0], data1[:, 1], data1[:, 2],
        data2[:, 0], data2[:, 2],
        data3[:, 1], data3[:, 2],
        data4[:, 0], data4[:, 1],
        data6[:, 0], data6[:, 2],
    ])
    idx_rel = jnp.concatenate([data2[:, 1], data3[:, 0], data6[:, 1]])

    tcomb = _build_combined(min_embedding.T, delta_embedding.T)
    trelc = jnp.concatenate([relation_embedding, scaling_embedding], axis=1)

    gcomb, grelc = _sc_gather_all(tcomb, trelc, idx_all, idx_rel)
    gcomb = gcomb.reshape(13, b, 2 * _DIM)
    grelc = grelc.reshape(3, b, 2 * _DIM)

    s = _compute_partials(gcomb, grelc)

    denom = float(b * _DIM)

    def l2s(j):
        s1 = s[6 + 2 * j]
        s2 = s[7 + 2 * j]
        return s1 / denom + jnp.maximum(jnp.sqrt(s2) - 1.0, 0.0)

    nf1_reg = l2s(0) + l2s(1)
    nf2_reg = l2s(2) + l2s(3) + l2s(4) + l2s(5)
    nf3_reg = l2s(6) + l2s(7) + l2s(8)
    nf4_reg = l2s(9) + l2s(10) + l2s(11)
    dj_reg = l2s(12) + l2s(13)
    nf3n_reg = l2s(14) + l2s(15) + l2s(16)
    return (s[0], s[1], s[2], s[3], s[4], s[5],
            nf1_reg, nf2_reg, nf3_reg, nf4_reg, dj_reg, nf3n_reg)
